# baseline (speedup 1.0000x reference)
import functools

import jax
import jax.numpy as jnp
from jax.experimental import pallas as pl
from jax.experimental.pallas import tpu as pltpu

_DP = 384


def _normalize_body(f_ref, u_ref):
    f = f_ref[0]
    n, d = f.shape
    ssq = jnp.sum(f * f, axis=-1, keepdims=True)
    inv = jax.lax.rsqrt(jnp.maximum(ssq, 1e-24))
    u = f * inv
    lane = jax.lax.broadcasted_iota(jnp.int32, (n, _DP), 1)
    aug = jnp.where(lane < d, 0.0, jnp.where(lane == d, 1.0, 0.0))
    padded = jnp.pad(u, ((0, 0), (0, _DP - d)))
    u_ref[0] = (padded + aug).astype(u_ref.dtype)


def _pair_body(ua_ref, ub_ref, sp_ref, dt_ref, cnt_ref, tot_ref):
    h = pl.program_id(1)

    @pl.when(h == 0)
    def _init():
        cnt_ref[0, 0, 0] = 0.0
        tot_ref[0, 0, 0] = 0.0

    sp = sp_ref[0]
    xs = sp[:, 0:1]
    ys = sp[:, 1:2]
    dt = dt_ref[0]
    xd = dt[0:1, :]
    yd = dt[1:2, :]

    xd2 = xd + xd
    yd2 = yd + yd
    rb = xd * xd + yd * yd - 1.0
    a2 = xs * xs + ys * ys
    lhs = xs * xd2 + ys * yd2
    rhs = a2 + rb
    maskf = (rhs <= lhs).astype(jnp.bfloat16)

    v = jax.lax.dot_general(
        maskf, ub_ref[0],
        dimension_numbers=(((1,), (0,)), ((), ())),
        preferred_element_type=jnp.float32)

    d = ua_ref.shape[2] - (_DP - 256)
    cnt = jnp.sum(v[:, d:d + 1])
    mcos = jnp.sum(v[:, :d] * ua_ref[0, :, :d].astype(jnp.float32))
    cnt_ref[0, 0, 0] += cnt
    tot_ref[0, 0, 0] += cnt - mcos


def kernel(features, pts_src, pts_dst, invis_idx, height, width):
    del invis_idx
    B, N, D = features.shape
    radius = 1.0
    fx = (jnp.asarray(width, jnp.float32) - 1.0) / 2.0
    fy = (jnp.asarray(height, jnp.float32) - 1.0) / 2.0
    factor = jnp.stack([fx, fy]) / radius

    src_pix = (pts_src + 1.0) * factor
    dst_pix = (pts_dst.reshape(B * B, N, 2) + 1.0) * factor

    src_p = jnp.pad(src_pix, ((0, 0), (0, 0), (0, 6)))
    dst_t = jnp.pad(jnp.transpose(dst_pix, (0, 2, 1)),
                    ((0, 0), (0, 6), (0, 0)))

    u = pl.pallas_call(
        _normalize_body,
        grid=(B,),
        in_specs=[pl.BlockSpec((1, N, D), lambda b: (b, 0, 0))],
        out_specs=pl.BlockSpec((1, N, _DP), lambda b: (b, 0, 0)),
        out_shape=jax.ShapeDtypeStruct((B, N, _DP), jnp.bfloat16),
    )(features)

    H = (B * B) // 2
    cnt, tot = pl.pallas_call(
        _pair_body,
        grid=(2, H),
        in_specs=[
            pl.BlockSpec((1, N, _DP), lambda c, h: ((c * H + h) % B, 0, 0)),
            pl.BlockSpec((1, N, _DP), lambda c, h: ((c * H + h) // B, 0, 0)),
            pl.BlockSpec((1, N, 8), lambda c, h: ((c * H + h) // B, 0, 0)),
            pl.BlockSpec((1, 8, N), lambda c, h: (c * H + h, 0, 0)),
        ],
        out_specs=[
            pl.BlockSpec((1, 1, 1), lambda c, h: (c, 0, 0),
                         memory_space=pltpu.SMEM),
            pl.BlockSpec((1, 1, 1), lambda c, h: (c, 0, 0),
                         memory_space=pltpu.SMEM),
        ],
        out_shape=[
            jax.ShapeDtypeStruct((2, 1, 1), jnp.float32),
            jax.ShapeDtypeStruct((2, 1, 1), jnp.float32),
        ],
        compiler_params=pltpu.CompilerParams(
            dimension_semantics=("parallel", "arbitrary")),
    )(u, u, src_p, dst_t)

    return jnp.sum(tot) / jnp.sum(cnt)


# speedup vs baseline: 1.4680x; 1.4680x over previous
import functools

import jax
import jax.numpy as jnp
from jax.experimental import pallas as pl
from jax.experimental.pallas import tpu as pltpu

_DP = 384


def _normalize_body(f_ref, u_ref):
    f = f_ref[0]
    n, d = f.shape
    ssq = jnp.sum(f * f, axis=-1, keepdims=True)
    inv = jax.lax.rsqrt(jnp.maximum(ssq, 1e-24))
    u = f * inv
    lane = jax.lax.broadcasted_iota(jnp.int32, (n, _DP), 1)
    aug = jnp.where(lane < d, 0.0, jnp.where(lane == d, 1.0, 0.0))
    padded = jnp.pad(u, ((0, 0), (0, _DP - d)))
    u_ref[0] = (padded + aug).astype(u_ref.dtype)


def _pair_body(ua_ref, ub_ref, sp_ref, dt_ref, cnt_ref, tot_ref):
    k = pl.program_id(0)

    @pl.when(k == 0)
    def _init():
        cnt_ref[0, 0] = 0.0
        tot_ref[0, 0] = 0.0

    sp = sp_ref[0]
    xs = sp[:, 0:1]
    ys = sp[:, 1:2]
    a2 = xs * xs + ys * ys

    def half_mask(dt):
        xd = dt[0:1, :]
        yd = dt[1:2, :]
        xd2 = xd + xd
        yd2 = yd + yd
        rb = xd * xd + yd * yd - 1.0
        return ((a2 + rb) <= (xs * xd2 + ys * yd2)).astype(jnp.bfloat16)

    m1 = half_mask(dt_ref[0])
    m2 = half_mask(dt_ref[1])

    ub = ub_ref[0]
    dims = (((1,), (0,)), ((), ()))
    v1 = jax.lax.dot_general(m1, ub, dims,
                             preferred_element_type=jnp.float32)
    v2 = jax.lax.dot_general(m2, ub, dims,
                             preferred_element_type=jnp.float32)

    d = 256
    cnt = jnp.sum(v1[:, d:d + 1]) + jnp.sum(v2[:, d:d + 1])
    mcos = (jnp.sum(v1[:, :d] * ua_ref[0, :, :d].astype(jnp.float32))
            + jnp.sum(v2[:, :d] * ua_ref[1, :, :d].astype(jnp.float32)))
    cnt_ref[0, 0] += cnt
    tot_ref[0, 0] += cnt - mcos


def kernel(features, pts_src, pts_dst, invis_idx, height, width):
    del invis_idx
    B, N, D = features.shape
    radius = 1.0
    fx = (jnp.asarray(width, jnp.float32) - 1.0) / 2.0
    fy = (jnp.asarray(height, jnp.float32) - 1.0) / 2.0
    factor = jnp.stack([fx, fy]) / radius

    src_pix = (pts_src + 1.0) * factor
    dst_pix = (pts_dst.reshape(B * B, N, 2) + 1.0) * factor

    src_p = jnp.pad(src_pix, ((0, 0), (0, 0), (0, 6)))
    dst_t = jnp.pad(jnp.transpose(dst_pix, (0, 2, 1)),
                    ((0, 0), (0, 6), (0, 0)))

    u = pl.pallas_call(
        _normalize_body,
        grid=(B,),
        in_specs=[pl.BlockSpec((1, N, D), lambda b: (b, 0, 0))],
        out_specs=pl.BlockSpec((1, N, _DP), lambda b: (b, 0, 0)),
        out_shape=jax.ShapeDtypeStruct((B, N, _DP), jnp.bfloat16),
    )(features)

    cnt, tot = pl.pallas_call(
        _pair_body,
        grid=(B * B // 2,),
        in_specs=[
            pl.BlockSpec((2, N, _DP), lambda k: (k % 4, 0, 0)),
            pl.BlockSpec((1, N, _DP), lambda k: (k // 4, 0, 0)),
            pl.BlockSpec((1, N, 8), lambda k: (k // 4, 0, 0)),
            pl.BlockSpec((2, 8, N), lambda k: (k, 0, 0)),
        ],
        out_specs=[
            pl.BlockSpec(memory_space=pltpu.SMEM),
            pl.BlockSpec(memory_space=pltpu.SMEM),
        ],
        out_shape=[
            jax.ShapeDtypeStruct((1, 1), jnp.float32),
            jax.ShapeDtypeStruct((1, 1), jnp.float32),
        ],
        compiler_params=pltpu.CompilerParams(
            dimension_semantics=("arbitrary",)),
    )(u, u, src_p, dst_t)

    return tot[0, 0] / cnt[0, 0]


# speedup vs baseline: 1.8051x; 1.2296x over previous
import functools

import jax
import jax.numpy as jnp
from jax.experimental import pallas as pl
from jax.experimental.pallas import tpu as pltpu

_DP = 384


def _normalize_body(f_ref, u_ref):
    f = f_ref[0]
    n, d = f.shape
    ssq = jnp.sum(f * f, axis=-1, keepdims=True)
    inv = jax.lax.rsqrt(jnp.maximum(ssq, 1e-24))
    u = f * inv
    lane = jax.lax.broadcasted_iota(jnp.int32, (n, _DP), 1)
    aug = jnp.where(lane < d, 0.0, jnp.where(lane == d, 1.0, 0.0))
    padded = jnp.pad(u, ((0, 0), (0, _DP - d)))
    u_ref[0] = (padded + aug).astype(u_ref.dtype)


def _pair_body(ua_ref, ub_ref, sp_ref, dt_ref, cnt_ref, tot_ref):
    k = pl.program_id(0)

    @pl.when(k == 0)
    def _init():
        cnt_ref[0, 0] = 0.0
        tot_ref[0, 0] = 0.0

    sp = sp_ref[0]
    xs = sp[:, 0:1]
    ys = sp[:, 1:2]
    a2 = xs * xs + ys * ys

    def half_mask(dt):
        xd = dt[0:1, :]
        yd = dt[1:2, :]
        xd2 = xd + xd
        yd2 = yd + yd
        rb = xd * xd + yd * yd - 1.0
        return ((a2 + rb) <= (xs * xd2 + ys * yd2)).astype(jnp.bfloat16)

    ub = ub_ref[0]
    dims = (((1,), (0,)), ((), ()))
    d = 256
    cnt = 0.0
    mcos = 0.0
    for p in range(dt_ref.shape[0]):
        m = half_mask(dt_ref[p])
        v = jax.lax.dot_general(m, ub, dims,
                                preferred_element_type=jnp.float32)
        cnt += jnp.sum(v[:, d:d + 1])
        mcos += jnp.sum(v[:, :d] * ua_ref[p, :, :d].astype(jnp.float32))
    cnt_ref[0, 0] += cnt
    tot_ref[0, 0] += cnt - mcos


def kernel(features, pts_src, pts_dst, invis_idx, height, width):
    del invis_idx
    B, N, D = features.shape
    radius = 1.0
    fx = (jnp.asarray(width, jnp.float32) - 1.0) / 2.0
    fy = (jnp.asarray(height, jnp.float32) - 1.0) / 2.0
    factor = jnp.stack([fx, fy]) / radius

    src_pix = (pts_src + 1.0) * factor
    dst_pix = (pts_dst.reshape(B * B, N, 2) + 1.0) * factor

    src_p = jnp.pad(src_pix, ((0, 0), (0, 0), (0, 6)))
    dst_t = jnp.pad(jnp.transpose(dst_pix, (0, 2, 1)),
                    ((0, 0), (0, 6), (0, 0)))

    u = pl.pallas_call(
        _normalize_body,
        grid=(B,),
        in_specs=[pl.BlockSpec((1, N, D), lambda b: (b, 0, 0))],
        out_specs=pl.BlockSpec((1, N, _DP), lambda b: (b, 0, 0)),
        out_shape=jax.ShapeDtypeStruct((B, N, _DP), jnp.bfloat16),
    )(features)

    cnt, tot = pl.pallas_call(
        _pair_body,
        grid=(B * B // 4,),
        in_specs=[
            pl.BlockSpec((4, N, _DP), lambda k: (k % 2, 0, 0)),
            pl.BlockSpec((1, N, _DP), lambda k: (k // 2, 0, 0)),
            pl.BlockSpec((1, N, 8), lambda k: (k // 2, 0, 0)),
            pl.BlockSpec((4, 8, N), lambda k: (k, 0, 0)),
        ],
        out_specs=[
            pl.BlockSpec(memory_space=pltpu.SMEM),
            pl.BlockSpec(memory_space=pltpu.SMEM),
        ],
        out_shape=[
            jax.ShapeDtypeStruct((1, 1), jnp.float32),
            jax.ShapeDtypeStruct((1, 1), jnp.float32),
        ],
        compiler_params=pltpu.CompilerParams(
            dimension_semantics=("arbitrary",)),
    )(u, u, src_p, dst_t)

    return tot[0, 0] / cnt[0, 0]


# speedup vs baseline: 2.0326x; 1.1261x over previous
import functools

import jax
import jax.numpy as jnp
from jax.experimental import pallas as pl
from jax.experimental.pallas import tpu as pltpu

_DP = 384


def _normalize_body(f_ref, u_ref):
    f = f_ref[0]
    n, d = f.shape
    ssq = jnp.sum(f * f, axis=-1, keepdims=True)
    inv = jax.lax.rsqrt(jnp.maximum(ssq, 1e-24))
    u = f * inv
    lane = jax.lax.broadcasted_iota(jnp.int32, (n, _DP), 1)
    aug = jnp.where(lane < d, 0.0, jnp.where(lane == d, 1.0, 0.0))
    padded = jnp.pad(u, ((0, 0), (0, _DP - d)))
    u_ref[0] = (padded + aug).astype(u_ref.dtype)


def _pair_body(ua_ref, ub_ref, sp_ref, dt_ref, cnt_ref, tot_ref):
    k = pl.program_id(0)

    @pl.when(k == 0)
    def _init():
        cnt_ref[0, 0] = 0.0
        tot_ref[0, 0] = 0.0

    sp = sp_ref[0]
    xs = sp[:, 0:1]
    ys = sp[:, 1:2]
    a2 = xs * xs + ys * ys

    def half_mask(dt):
        xd = dt[0:1, :]
        yd = dt[1:2, :]
        xd2 = xd + xd
        yd2 = yd + yd
        rb = xd * xd + yd * yd - 1.0
        return ((a2 + rb) <= (xs * xd2 + ys * yd2)).astype(jnp.bfloat16)

    ub = ub_ref[0]
    dims = (((1,), (0,)), ((), ()))
    d = 256
    cnt = 0.0
    mcos = 0.0
    for p in range(dt_ref.shape[0]):
        m = half_mask(dt_ref[p])
        v = jax.lax.dot_general(m, ub, dims,
                                preferred_element_type=jnp.float32)
        cnt += jnp.sum(v[:, d:d + 1])
        mcos += jnp.sum(v[:, :d] * ua_ref[p, :, :d].astype(jnp.float32))
    cnt_ref[0, 0] += cnt
    tot_ref[0, 0] += cnt - mcos


def kernel(features, pts_src, pts_dst, invis_idx, height, width):
    del invis_idx
    B, N, D = features.shape
    radius = 1.0
    fx = (jnp.asarray(width, jnp.float32) - 1.0) / 2.0
    fy = (jnp.asarray(height, jnp.float32) - 1.0) / 2.0
    factor = jnp.stack([fx, fy]) / radius

    src_pix = (pts_src + 1.0) * factor
    dst_pix = (pts_dst.reshape(B * B, N, 2) + 1.0) * factor

    src_p = jnp.pad(src_pix, ((0, 0), (0, 0), (0, 6)))
    dst_t = jnp.pad(jnp.transpose(dst_pix, (0, 2, 1)),
                    ((0, 0), (0, 6), (0, 0)))

    u = pl.pallas_call(
        _normalize_body,
        grid=(B,),
        in_specs=[pl.BlockSpec((1, N, D), lambda b: (b, 0, 0))],
        out_specs=pl.BlockSpec((1, N, _DP), lambda b: (b, 0, 0)),
        out_shape=jax.ShapeDtypeStruct((B, N, _DP), jnp.bfloat16),
    )(features)

    cnt, tot = pl.pallas_call(
        _pair_body,
        grid=(B,),
        in_specs=[
            pl.BlockSpec((B, N, _DP), lambda k: (0, 0, 0)),
            pl.BlockSpec((1, N, _DP), lambda k: (k, 0, 0)),
            pl.BlockSpec((1, N, 8), lambda k: (k, 0, 0)),
            pl.BlockSpec((B, 8, N), lambda k: (k, 0, 0)),
        ],
        out_specs=[
            pl.BlockSpec(memory_space=pltpu.SMEM),
            pl.BlockSpec(memory_space=pltpu.SMEM),
        ],
        out_shape=[
            jax.ShapeDtypeStruct((1, 1), jnp.float32),
            jax.ShapeDtypeStruct((1, 1), jnp.float32),
        ],
        compiler_params=pltpu.CompilerParams(
            dimension_semantics=("arbitrary",)),
    )(u, u, src_p, dst_t)

    return tot[0, 0] / cnt[0, 0]
